# baseline (speedup 1.0000x reference)
import functools

import jax
import jax.numpy as jnp
from jax import lax
from jax.experimental import pallas as pl
from jax.experimental.pallas import tpu as pltpu
from jax.experimental.pallas import tpu_sc as plsc

_N = 10000
_NP = 10240
_E = 320000
_EP = 327680
_D = 128
_G = 64
_NC = 2
_NS = 16
_CH = 128
_EPW = _EP // (_NC * _NS)
_RPT = _NP // _NS
_ZBASE = _NP - _CH
_BN = 1024
_EPS = 1e-5


_NB = 4
_CPW = _EPW // _CH
_C0 = 40


def _sc_segment_sum(xp, sd3):
    mesh = plsc.VectorSubcoreMesh(core_axis_name="c", subcore_axis_name="s")

    @functools.partial(
        pl.kernel,
        mesh=mesh,
        out_type=jax.ShapeDtypeStruct((_NC * _NP, _D), jnp.float32),
        scratch_types=[
            pltpu.VMEM((_NB, 2, _CH), jnp.int32),
            pltpu.VMEM((2, _CH, _D), jnp.float32),
            pltpu.VMEM_SHARED((_NP, _D), jnp.float32),
            pltpu.SemaphoreType.DMA,
            pltpu.SemaphoreType.DMA,
            pltpu.SemaphoreType.DMA,
            pltpu.SemaphoreType.DMA,
            pltpu.SemaphoreType.DMA,
            pltpu.SemaphoreType.DMA,
        ],
    )
    def _sc(x_hbm, sd_hbm, out_hbm, sd, rows, acc,
            i0, i1, i2, i3, g0, g1):
        cid = lax.axis_index("c")
        sid = lax.axis_index("s")
        isem = (i0, i1, i2, i3)
        gsem = (g0, g1)
        cbase = sid * (2 * _CPW) + jnp.where(cid == 0, 0, _C0)
        nch = jnp.where(cid == 0, _C0, 2 * _CPW - _C0)

        pltpu.sync_copy(x_hbm.at[pl.ds(_ZBASE, _CH)], rows.at[0])

        def _zero(j, c):
            pltpu.sync_copy(rows.at[0],
                            acc.at[pl.ds(sid * _RPT + j * _CH, _CH)])
            return c

        lax.fori_loop(0, _RPT // _CH, _zero, 0)
        plsc.subcore_barrier()

        def _ifire(c, k):
            pltpu.async_copy(sd_hbm.at[cbase + c], sd.at[k], isem[k])

        def _iwait(c, k):
            pltpu.make_async_copy(sd_hbm.at[cbase + c], sd.at[k],
                                  isem[k]).wait()

        def _gfire(ki, kr):
            pltpu.async_copy(x_hbm.at[sd.at[ki, 0]], rows.at[kr], gsem[kr])

        def _gwait(ki, kr):
            pltpu.make_async_copy(x_hbm.at[sd.at[ki, 0]], rows.at[kr],
                                  gsem[kr]).wait()

        _ifire(0, 0)
        _ifire(1, 1)
        _ifire(2, 2)
        _iwait(0, 0)
        _gfire(0, 0)

        def _step(j, carry):
            for k in range(_NB):
                c = j * _NB + k
                k3 = (k + 3) % _NB
                k1 = (k + 1) % _NB
                kr = k % 2

                @pl.when(c + 3 < nch)
                def _():
                    _ifire(c + 3, k3)

                @pl.when(c + 1 < nch)
                def _():
                    _iwait(c + 1, k1)
                    _gfire(k1, 1 - kr)

                _gwait(k, kr)
                pltpu.sync_copy(rows.at[kr], acc.at[sd.at[k, 1]], add=True)
            return carry

        lax.fori_loop(0, nch // _NB, _step, 0)
        plsc.subcore_barrier()

        def _out(j, c):
            r0 = sid * _RPT + j * _CH
            pltpu.sync_copy(acc.at[pl.ds(r0, _CH)], rows.at[0])
            pltpu.sync_copy(rows.at[0],
                            out_hbm.at[pl.ds(cid * _NP + r0, _CH)])
            return c

        lax.fori_loop(0, _RPT // _CH, _out, 0)

    return _sc(xp, sd3)


def _row_mask(i):
    return i * _BN + lax.broadcasted_iota(jnp.int32, (_BN, _D), 0) < _N


def _mlp_body(x_ref, a0_ref, a1_ref, w1_ref, b1_ref, w2_ref, b2_ref,
              h_ref, st_ref):
    i = pl.program_id(0)
    xin = x_ref[...] + a0_ref[...] + a1_ref[...]
    t = jnp.dot(xin, w1_ref[...],
                preferred_element_type=jnp.float32) + b1_ref[...]
    t = jnp.maximum(t, 0.0)
    u = jnp.dot(t, w2_ref[...],
                preferred_element_type=jnp.float32) + b2_ref[...]
    h = jnp.where(_row_mask(i), jnp.maximum(u, 0.0), 0.0)
    h_ref[...] = h
    s = jnp.sum(h, axis=0, keepdims=True)
    ss = jnp.sum(h * h, axis=0, keepdims=True)
    st = jnp.concatenate([s, ss, jnp.zeros((6, _D), jnp.float32)], axis=0)

    @pl.when(i == 0)
    def _():
        st_ref[...] = st

    @pl.when(i > 0)
    def _():
        st_ref[...] = st_ref[...] + st


def _mlp(xp, agg, w1, b1, w2, b2):
    nb = _NP // _BN
    return pl.pallas_call(
        _mlp_body,
        grid=(nb,),
        in_specs=[
            pl.BlockSpec((_BN, _D), lambda i: (i, 0)),
            pl.BlockSpec((_BN, _D), lambda i: (i, 0)),
            pl.BlockSpec((_BN, _D), lambda i: (i + nb, 0)),
            pl.BlockSpec((_D, _D), lambda i: (0, 0)),
            pl.BlockSpec((1, _D), lambda i: (0, 0)),
            pl.BlockSpec((_D, _D), lambda i: (0, 0)),
            pl.BlockSpec((1, _D), lambda i: (0, 0)),
        ],
        out_specs=[
            pl.BlockSpec((_BN, _D), lambda i: (i, 0)),
            pl.BlockSpec((8, _D), lambda i: (0, 0)),
        ],
        out_shape=[
            jax.ShapeDtypeStruct((_NP, _D), jnp.float32),
            jax.ShapeDtypeStruct((8, _D), jnp.float32),
        ],
    )(xp, agg, agg, w1, b1.reshape(1, _D), w2, b2.reshape(1, _D))


def _affine(st_ref, g_ref, be_ref):
    mean = st_ref[0:1, :] * (1.0 / _N)
    var = st_ref[1:2, :] * (1.0 / _N) - mean * mean
    a = g_ref[...] * lax.rsqrt(var + _EPS)
    b = be_ref[...] - mean * a
    return a, b


def _norm_body(h_ref, st_ref, g_ref, be_ref, o_ref):
    i = pl.program_id(0)
    a, b = _affine(st_ref, g_ref, be_ref)
    o_ref[...] = jnp.where(_row_mask(i), h_ref[...] * a + b, 0.0)


def _norm(h, st, gamma, beta):
    return pl.pallas_call(
        _norm_body,
        grid=(_NP // _BN,),
        in_specs=[
            pl.BlockSpec((_BN, _D), lambda i: (i, 0)),
            pl.BlockSpec((8, _D), lambda i: (0, 0)),
            pl.BlockSpec((1, _D), lambda i: (0, 0)),
            pl.BlockSpec((1, _D), lambda i: (0, 0)),
        ],
        out_specs=pl.BlockSpec((_BN, _D), lambda i: (i, 0)),
        out_shape=jax.ShapeDtypeStruct((_NP, _D), jnp.float32),
    )(h, st, gamma.reshape(1, _D), beta.reshape(1, _D))


def _pool_body(h_ref, st_ref, g_ref, be_ref, seg_ref, o_ref):
    i = pl.program_id(0)
    a, b = _affine(st_ref, g_ref, be_ref)
    hn = jnp.where(_row_mask(i), h_ref[...] * a + b, 0.0)
    seg = seg_ref[0, 0, :]
    onehot = (lax.broadcasted_iota(jnp.int32, (_G, _BN), 0)
              == seg[None, :]).astype(jnp.float32)
    c = jnp.dot(onehot, hn, preferred_element_type=jnp.float32,
                precision=lax.Precision.HIGHEST)

    @pl.when(i == 0)
    def _():
        o_ref[...] = c

    @pl.when(i > 0)
    def _():
        o_ref[...] = o_ref[...] + c


def _pool(h, st, gamma, beta, seg3):
    return pl.pallas_call(
        _pool_body,
        grid=(_NP // _BN,),
        in_specs=[
            pl.BlockSpec((_BN, _D), lambda i: (i, 0)),
            pl.BlockSpec((8, _D), lambda i: (0, 0)),
            pl.BlockSpec((1, _D), lambda i: (0, 0)),
            pl.BlockSpec((1, _D), lambda i: (0, 0)),
            pl.BlockSpec((1, 1, _BN), lambda i: (i, 0, 0)),
        ],
        out_specs=pl.BlockSpec((_G, _D), lambda i: (0, 0)),
        out_shape=jax.ShapeDtypeStruct((_G, _D), jnp.float32),
    )(h, st, gamma.reshape(1, _D), beta.reshape(1, _D), seg3)


def kernel(x, edge_index, batch,
           W1_1, b1_1, W1_2, b1_2, gamma1, beta1,
           W2_1, b2_1, W2_2, b2_2, gamma2, beta2,
           W3_1, b3_1, W3_2, b3_2, gamma3, beta3):
    f32 = jnp.float32
    src = edge_index[0]
    dst = edge_index[1]
    epad = jnp.full((2, _EP - _E), _N, jnp.int32)
    sd3 = jnp.concatenate([edge_index, epad], axis=1)
    sd3 = sd3.reshape(2, _EP // _CH, _CH).transpose(1, 0, 2)
    xp = jnp.concatenate([x.astype(f32), jnp.zeros((_NP - _N, _D), f32)])
    seg3 = jnp.concatenate([batch, jnp.zeros((_NP - _N,), jnp.int32)])
    seg3 = seg3.reshape(_NP // _BN, 1, _BN)

    agg1 = _sc_segment_sum(xp, sd3)
    h1, st1 = _mlp(xp, agg1, W1_1, b1_1, W1_2, b1_2)
    h1n = _norm(h1, st1, gamma1, beta1)
    agg2 = _sc_segment_sum(h1n, sd3)
    h2, st2 = _mlp(h1n, agg2, W2_1, b2_1, W2_2, b2_2)
    h2n = _norm(h2, st2, gamma2, beta2)
    agg3 = _sc_segment_sum(h2n, sd3)
    h3, st3 = _mlp(h2n, agg3, W3_1, b3_1, W3_2, b3_2)
    return _pool(h3, st3, gamma3, beta3, seg3)


# speedup vs baseline: 1.0549x; 1.0549x over previous
import functools

import jax
import jax.numpy as jnp
from jax import lax
from jax.experimental import pallas as pl
from jax.experimental.pallas import tpu as pltpu
from jax.experimental.pallas import tpu_sc as plsc

_N = 10000
_NP = 10240
_E = 320000
_EP = 327680
_D = 128
_G = 64
_NC = 2
_NS = 16
_CH = 128
_EPW = _EP // (_NC * _NS)
_RPT = _NP // _NS
_ZBASE = _NP - _CH
_BN = 1024
_EPS = 1e-5


_NB = 4
_CPW = _EPW // _CH
_C0 = 108


def _sc_segment_sum(xp, sd3):
    mesh = plsc.VectorSubcoreMesh(core_axis_name="c", subcore_axis_name="s")

    @functools.partial(
        pl.kernel,
        mesh=mesh,
        out_type=jax.ShapeDtypeStruct((_NC * _NP, _D), jnp.float32),
        scratch_types=[
            pltpu.VMEM((_NB, 2, _CH), jnp.int32),
            pltpu.VMEM((2, _CH, _D), jnp.float32),
            pltpu.VMEM_SHARED((_NP, _D), jnp.float32),
            pltpu.SemaphoreType.DMA,
            pltpu.SemaphoreType.DMA,
            pltpu.SemaphoreType.DMA,
            pltpu.SemaphoreType.DMA,
            pltpu.SemaphoreType.DMA,
            pltpu.SemaphoreType.DMA,
        ],
    )
    def _sc(x_hbm, sd_hbm, out_hbm, sd, rows, acc,
            i0, i1, i2, i3, g0, g1):
        cid = lax.axis_index("c")
        sid = lax.axis_index("s")
        isem = (i0, i1, i2, i3)
        gsem = (g0, g1)
        cbase = sid * (2 * _CPW) + jnp.where(cid == 0, 0, _C0)
        nch = jnp.where(cid == 0, _C0, 2 * _CPW - _C0)

        pltpu.sync_copy(x_hbm.at[pl.ds(_ZBASE, _CH)], rows.at[0])

        def _zero(j, c):
            pltpu.sync_copy(rows.at[0],
                            acc.at[pl.ds(sid * _RPT + j * _CH, _CH)])
            return c

        lax.fori_loop(0, _RPT // _CH, _zero, 0)
        plsc.subcore_barrier()

        def _ifire(c, k):
            pltpu.async_copy(sd_hbm.at[cbase + c], sd.at[k], isem[k])

        def _iwait(c, k):
            pltpu.make_async_copy(sd_hbm.at[cbase + c], sd.at[k],
                                  isem[k]).wait()

        def _gfire(ki, kr):
            pltpu.async_copy(x_hbm.at[sd.at[ki, 0]], rows.at[kr], gsem[kr])

        def _gwait(ki, kr):
            pltpu.make_async_copy(x_hbm.at[sd.at[ki, 0]], rows.at[kr],
                                  gsem[kr]).wait()

        _ifire(0, 0)
        _ifire(1, 1)
        _ifire(2, 2)
        _iwait(0, 0)
        _gfire(0, 0)

        def _step(j, carry):
            for k in range(_NB):
                c = j * _NB + k
                k3 = (k + 3) % _NB
                k1 = (k + 1) % _NB
                kr = k % 2

                @pl.when(c + 3 < nch)
                def _():
                    _ifire(c + 3, k3)

                @pl.when(c + 1 < nch)
                def _():
                    _iwait(c + 1, k1)
                    _gfire(k1, 1 - kr)

                _gwait(k, kr)
                pltpu.sync_copy(rows.at[kr], acc.at[sd.at[k, 1]], add=True)
            return carry

        lax.fori_loop(0, nch // _NB, _step, 0)
        plsc.subcore_barrier()

        def _out(j, c):
            r0 = sid * _RPT + j * _CH
            pltpu.sync_copy(acc.at[pl.ds(r0, _CH)], rows.at[0])
            pltpu.sync_copy(rows.at[0],
                            out_hbm.at[pl.ds(cid * _NP + r0, _CH)])
            return c

        lax.fori_loop(0, _RPT // _CH, _out, 0)

    return _sc(xp, sd3)


def _row_mask(i):
    return i * _BN + lax.broadcasted_iota(jnp.int32, (_BN, _D), 0) < _N


def _mlp_body(x_ref, a0_ref, a1_ref, w1_ref, b1_ref, w2_ref, b2_ref,
              h_ref, st_ref):
    i = pl.program_id(0)
    xin = x_ref[...] + a0_ref[...] + a1_ref[...]
    t = jnp.dot(xin, w1_ref[...],
                preferred_element_type=jnp.float32) + b1_ref[...]
    t = jnp.maximum(t, 0.0)
    u = jnp.dot(t, w2_ref[...],
                preferred_element_type=jnp.float32) + b2_ref[...]
    h = jnp.where(_row_mask(i), jnp.maximum(u, 0.0), 0.0)
    h_ref[...] = h
    s = jnp.sum(h, axis=0, keepdims=True)
    ss = jnp.sum(h * h, axis=0, keepdims=True)
    st = jnp.concatenate([s, ss, jnp.zeros((6, _D), jnp.float32)], axis=0)

    @pl.when(i == 0)
    def _():
        st_ref[...] = st

    @pl.when(i > 0)
    def _():
        st_ref[...] = st_ref[...] + st


def _mlp(xp, agg, w1, b1, w2, b2):
    nb = _NP // _BN
    return pl.pallas_call(
        _mlp_body,
        grid=(nb,),
        in_specs=[
            pl.BlockSpec((_BN, _D), lambda i: (i, 0)),
            pl.BlockSpec((_BN, _D), lambda i: (i, 0)),
            pl.BlockSpec((_BN, _D), lambda i: (i + nb, 0)),
            pl.BlockSpec((_D, _D), lambda i: (0, 0)),
            pl.BlockSpec((1, _D), lambda i: (0, 0)),
            pl.BlockSpec((_D, _D), lambda i: (0, 0)),
            pl.BlockSpec((1, _D), lambda i: (0, 0)),
        ],
        out_specs=[
            pl.BlockSpec((_BN, _D), lambda i: (i, 0)),
            pl.BlockSpec((8, _D), lambda i: (0, 0)),
        ],
        out_shape=[
            jax.ShapeDtypeStruct((_NP, _D), jnp.float32),
            jax.ShapeDtypeStruct((8, _D), jnp.float32),
        ],
    )(xp, agg, agg, w1, b1.reshape(1, _D), w2, b2.reshape(1, _D))


def _affine(st_ref, g_ref, be_ref):
    mean = st_ref[0:1, :] * (1.0 / _N)
    var = st_ref[1:2, :] * (1.0 / _N) - mean * mean
    a = g_ref[...] * lax.rsqrt(var + _EPS)
    b = be_ref[...] - mean * a
    return a, b


def _norm_body(h_ref, st_ref, g_ref, be_ref, o_ref):
    i = pl.program_id(0)
    a, b = _affine(st_ref, g_ref, be_ref)
    o_ref[...] = jnp.where(_row_mask(i), h_ref[...] * a + b, 0.0)


def _norm(h, st, gamma, beta):
    return pl.pallas_call(
        _norm_body,
        grid=(_NP // _BN,),
        in_specs=[
            pl.BlockSpec((_BN, _D), lambda i: (i, 0)),
            pl.BlockSpec((8, _D), lambda i: (0, 0)),
            pl.BlockSpec((1, _D), lambda i: (0, 0)),
            pl.BlockSpec((1, _D), lambda i: (0, 0)),
        ],
        out_specs=pl.BlockSpec((_BN, _D), lambda i: (i, 0)),
        out_shape=jax.ShapeDtypeStruct((_NP, _D), jnp.float32),
    )(h, st, gamma.reshape(1, _D), beta.reshape(1, _D))


def _pool_body(h_ref, st_ref, g_ref, be_ref, seg_ref, o_ref):
    i = pl.program_id(0)
    a, b = _affine(st_ref, g_ref, be_ref)
    hn = jnp.where(_row_mask(i), h_ref[...] * a + b, 0.0)
    seg = seg_ref[0, 0, :]
    onehot = (lax.broadcasted_iota(jnp.int32, (_G, _BN), 0)
              == seg[None, :]).astype(jnp.float32)
    c = jnp.dot(onehot, hn, preferred_element_type=jnp.float32,
                precision=lax.Precision.HIGHEST)

    @pl.when(i == 0)
    def _():
        o_ref[...] = c

    @pl.when(i > 0)
    def _():
        o_ref[...] = o_ref[...] + c


def _pool(h, st, gamma, beta, seg3):
    return pl.pallas_call(
        _pool_body,
        grid=(_NP // _BN,),
        in_specs=[
            pl.BlockSpec((_BN, _D), lambda i: (i, 0)),
            pl.BlockSpec((8, _D), lambda i: (0, 0)),
            pl.BlockSpec((1, _D), lambda i: (0, 0)),
            pl.BlockSpec((1, _D), lambda i: (0, 0)),
            pl.BlockSpec((1, 1, _BN), lambda i: (i, 0, 0)),
        ],
        out_specs=pl.BlockSpec((_G, _D), lambda i: (0, 0)),
        out_shape=jax.ShapeDtypeStruct((_G, _D), jnp.float32),
    )(h, st, gamma.reshape(1, _D), beta.reshape(1, _D), seg3)


def kernel(x, edge_index, batch,
           W1_1, b1_1, W1_2, b1_2, gamma1, beta1,
           W2_1, b2_1, W2_2, b2_2, gamma2, beta2,
           W3_1, b3_1, W3_2, b3_2, gamma3, beta3):
    f32 = jnp.float32
    src = edge_index[0]
    dst = edge_index[1]
    epad = jnp.full((2, _EP - _E), _N, jnp.int32)
    sd3 = jnp.concatenate([edge_index, epad], axis=1)
    sd3 = sd3.reshape(2, _EP // _CH, _CH).transpose(1, 0, 2)
    xp = jnp.concatenate([x.astype(f32), jnp.zeros((_NP - _N, _D), f32)])
    seg3 = jnp.concatenate([batch, jnp.zeros((_NP - _N,), jnp.int32)])
    seg3 = seg3.reshape(_NP // _BN, 1, _BN)

    agg1 = _sc_segment_sum(xp, sd3)
    h1, st1 = _mlp(xp, agg1, W1_1, b1_1, W1_2, b1_2)
    h1n = _norm(h1, st1, gamma1, beta1)
    agg2 = _sc_segment_sum(h1n, sd3)
    h2, st2 = _mlp(h1n, agg2, W2_1, b2_1, W2_2, b2_2)
    h2n = _norm(h2, st2, gamma2, beta2)
    agg3 = _sc_segment_sum(h2n, sd3)
    h3, st3 = _mlp(h2n, agg3, W3_1, b3_1, W3_2, b3_2)
    return _pool(h3, st3, gamma3, beta3, seg3)


# speedup vs baseline: 1.0572x; 1.0022x over previous
import functools

import jax
import jax.numpy as jnp
from jax import lax
from jax.experimental import pallas as pl
from jax.experimental.pallas import tpu as pltpu
from jax.experimental.pallas import tpu_sc as plsc

_N = 10000
_NP = 10240
_E = 320000
_EP = 327680
_D = 128
_G = 64
_NC = 2
_NS = 16
_CH = 128
_EPW = _EP // (_NC * _NS)
_RPT = _NP // _NS
_ZBASE = _NP - _CH
_BN = 1024
_EPS = 1e-5


_NB = 4
_CPW = _EPW // _CH
_C0 = 108


def _sc_segment_sum(xb, sd3, zr):
    mesh = plsc.VectorSubcoreMesh(core_axis_name="c", subcore_axis_name="s")

    @functools.partial(
        pl.kernel,
        mesh=mesh,
        compiler_params=pltpu.CompilerParams(use_tc_tiling_on_sc=False),
        out_type=jax.ShapeDtypeStruct((_NC * _NP, _D), jnp.float32),
        scratch_types=[
            pltpu.VMEM((_NB, 2, _CH), jnp.int32),
            pltpu.VMEM((2, _CH, _D), jnp.float32),
            pltpu.VMEM_SHARED((_NP, _D), jnp.float32),
            pltpu.SemaphoreType.DMA,
            pltpu.SemaphoreType.DMA,
            pltpu.SemaphoreType.DMA,
            pltpu.SemaphoreType.DMA,
            pltpu.SemaphoreType.DMA,
            pltpu.SemaphoreType.DMA,
        ],
    )
    def _sc(x_hbm, sd_hbm, z_hbm, out_hbm, sd, rows, acc,
            i0, i1, i2, i3, g0, g1):
        cid = lax.axis_index("c")
        sid = lax.axis_index("s")
        isem = (i0, i1, i2, i3)
        gsem = (g0, g1)
        cbase = sid * (2 * _CPW) + jnp.where(cid == 0, 0, _C0)
        nch = jnp.where(cid == 0, _C0, 2 * _CPW - _C0)

        pltpu.sync_copy(z_hbm, rows.at[0])

        def _zero(j, c):
            pltpu.sync_copy(rows.at[0],
                            acc.at[pl.ds(sid * _RPT + j * _CH, _CH)])
            return c

        lax.fori_loop(0, _RPT // _CH, _zero, 0)
        plsc.subcore_barrier()

        def _ifire(c, k):
            pltpu.async_copy(sd_hbm.at[cbase + c], sd.at[k], isem[k])

        def _iwait(c, k):
            pltpu.make_async_copy(sd_hbm.at[cbase + c], sd.at[k],
                                  isem[k]).wait()

        def _gfire(ki, kr):
            pltpu.async_copy(x_hbm.at[sd.at[ki, 0]], rows.at[kr], gsem[kr])

        def _gwait(ki, kr):
            pltpu.make_async_copy(x_hbm.at[sd.at[ki, 0]], rows.at[kr],
                                  gsem[kr]).wait()

        m7fff = jnp.int32(0x7FFF)
        m8000 = jnp.int32(0x8000)
        msign = jnp.int32(-2147483648)
        bias = jnp.int32(112 << 10)

        def _f16up(mag, sgn):
            bits = lax.bitwise_or(sgn, lax.shift_left(mag + bias, 13))
            return jnp.where(mag == 0, 0.0,
                             lax.bitcast_convert_type(bits, jnp.float32))

        def _widen(kr):
            def _rows2(r2, c):
                for rr in range(2):
                    r = r2 * 2 + rr
                    for q in range(_D // 32):
                        u = rows[kr, r, pl.ds(q * 16, 16)]
                        lo = _f16up(
                            lax.bitwise_and(u, m7fff),
                            lax.shift_left(lax.bitwise_and(u, m8000), 16))
                        hi = _f16up(
                            lax.bitwise_and(
                                lax.shift_right_logical(u, 16), m7fff),
                            lax.bitwise_and(u, msign))
                        rf[r, pl.ds(q * 32, 16)] = lo
                        rf[r, pl.ds(q * 32 + 16, 16)] = hi
                return c

            lax.fori_loop(0, _CH // 2, _rows2, 0)

        _ifire(0, 0)
        _ifire(1, 1)
        _ifire(2, 2)
        _iwait(0, 0)
        _gfire(0, 0)

        def _step(j, carry):
            for k in range(_NB):
                c = j * _NB + k
                k3 = (k + 3) % _NB
                k1 = (k + 1) % _NB
                kr = k % 2

                @pl.when(c + 3 < nch)
                def _():
                    _ifire(c + 3, k3)

                @pl.when(c + 1 < nch)
                def _():
                    _iwait(c + 1, k1)
                    _gfire(k1, 1 - kr)

                _gwait(k, kr)
                pltpu.sync_copy(rows.at[kr], acc.at[sd.at[k, 1]], add=True)
            return carry

        lax.fori_loop(0, nch // _NB, _step, 0)
        plsc.subcore_barrier()

        def _out(j, c):
            r0 = sid * _RPT + j * _CH
            pltpu.sync_copy(acc.at[pl.ds(r0, _CH)], rows.at[0])
            pltpu.sync_copy(rows.at[0],
                            out_hbm.at[pl.ds(cid * _NP + r0, _CH)])
            return c

        lax.fori_loop(0, _RPT // _CH, _out, 0)

    return _sc(xb, sd3, zr)


def _row_mask(i):
    return i * _BN + lax.broadcasted_iota(jnp.int32, (_BN, _D), 0) < _N


def _mlp_body(x_ref, a0_ref, a1_ref, w1_ref, b1_ref, w2_ref, b2_ref,
              h_ref, st_ref):
    i = pl.program_id(0)
    xin = x_ref[...] + a0_ref[...] + a1_ref[...]
    t = jnp.dot(xin, w1_ref[...],
                preferred_element_type=jnp.float32) + b1_ref[...]
    t = jnp.maximum(t, 0.0)
    u = jnp.dot(t, w2_ref[...],
                preferred_element_type=jnp.float32) + b2_ref[...]
    h = jnp.where(_row_mask(i), jnp.maximum(u, 0.0), 0.0)
    h_ref[...] = h
    s = jnp.sum(h, axis=0, keepdims=True)
    ss = jnp.sum(h * h, axis=0, keepdims=True)
    st = jnp.concatenate([s, ss, jnp.zeros((6, _D), jnp.float32)], axis=0)

    @pl.when(i == 0)
    def _():
        st_ref[...] = st

    @pl.when(i > 0)
    def _():
        st_ref[...] = st_ref[...] + st


def _mlp(xp, agg, w1, b1, w2, b2):
    nb = _NP // _BN
    return pl.pallas_call(
        _mlp_body,
        grid=(nb,),
        in_specs=[
            pl.BlockSpec((_BN, _D), lambda i: (i, 0)),
            pl.BlockSpec((_BN, _D), lambda i: (i, 0)),
            pl.BlockSpec((_BN, _D), lambda i: (i + nb, 0)),
            pl.BlockSpec((_D, _D), lambda i: (0, 0)),
            pl.BlockSpec((1, _D), lambda i: (0, 0)),
            pl.BlockSpec((_D, _D), lambda i: (0, 0)),
            pl.BlockSpec((1, _D), lambda i: (0, 0)),
        ],
        out_specs=[
            pl.BlockSpec((_BN, _D), lambda i: (i, 0)),
            pl.BlockSpec((8, _D), lambda i: (0, 0)),
        ],
        out_shape=[
            jax.ShapeDtypeStruct((_NP, _D), jnp.float32),
            jax.ShapeDtypeStruct((8, _D), jnp.float32),
        ],
    )(xp, agg, agg, w1, b1.reshape(1, _D), w2, b2.reshape(1, _D))


def _affine(st_ref, g_ref, be_ref):
    mean = st_ref[0:1, :] * (1.0 / _N)
    var = st_ref[1:2, :] * (1.0 / _N) - mean * mean
    a = g_ref[...] * lax.rsqrt(var + _EPS)
    b = be_ref[...] - mean * a
    return a, b


def _norm_body(h_ref, st_ref, g_ref, be_ref, o_ref):
    i = pl.program_id(0)
    a, b = _affine(st_ref, g_ref, be_ref)
    o_ref[...] = jnp.where(_row_mask(i), h_ref[...] * a + b, 0.0)


def _norm(h, st, gamma, beta):
    return pl.pallas_call(
        _norm_body,
        grid=(_NP // _BN,),
        in_specs=[
            pl.BlockSpec((_BN, _D), lambda i: (i, 0)),
            pl.BlockSpec((8, _D), lambda i: (0, 0)),
            pl.BlockSpec((1, _D), lambda i: (0, 0)),
            pl.BlockSpec((1, _D), lambda i: (0, 0)),
        ],
        out_specs=pl.BlockSpec((_BN, _D), lambda i: (i, 0)),
        out_shape=jax.ShapeDtypeStruct((_NP, _D), jnp.float32),
    )(h, st, gamma.reshape(1, _D), beta.reshape(1, _D))


def _pool_body(h_ref, st_ref, g_ref, be_ref, seg_ref, o_ref):
    i = pl.program_id(0)
    a, b = _affine(st_ref, g_ref, be_ref)
    hn = jnp.where(_row_mask(i), h_ref[...] * a + b, 0.0)
    seg = seg_ref[0, 0, :]
    onehot = (lax.broadcasted_iota(jnp.int32, (_G, _BN), 0)
              == seg[None, :]).astype(jnp.float32)
    c = jnp.dot(onehot, hn, preferred_element_type=jnp.float32,
                precision=lax.Precision.HIGHEST)

    @pl.when(i == 0)
    def _():
        o_ref[...] = c

    @pl.when(i > 0)
    def _():
        o_ref[...] = o_ref[...] + c


def _pool(h, st, gamma, beta, seg3):
    return pl.pallas_call(
        _pool_body,
        grid=(_NP // _BN,),
        in_specs=[
            pl.BlockSpec((_BN, _D), lambda i: (i, 0)),
            pl.BlockSpec((8, _D), lambda i: (0, 0)),
            pl.BlockSpec((1, _D), lambda i: (0, 0)),
            pl.BlockSpec((1, _D), lambda i: (0, 0)),
            pl.BlockSpec((1, 1, _BN), lambda i: (i, 0, 0)),
        ],
        out_specs=pl.BlockSpec((_G, _D), lambda i: (0, 0)),
        out_shape=jax.ShapeDtypeStruct((_G, _D), jnp.float32),
    )(h, st, gamma.reshape(1, _D), beta.reshape(1, _D), seg3)


def kernel(x, edge_index, batch,
           W1_1, b1_1, W1_2, b1_2, gamma1, beta1,
           W2_1, b2_1, W2_2, b2_2, gamma2, beta2,
           W3_1, b3_1, W3_2, b3_2, gamma3, beta3):
    f32 = jnp.float32
    src = edge_index[0]
    dst = edge_index[1]
    epad = jnp.full((2, _EP - _E), _N, jnp.int32)
    sd3 = jnp.concatenate([edge_index, epad], axis=1)
    sd3 = sd3.reshape(2, _EP // _CH, _CH).transpose(1, 0, 2)
    xp = jnp.concatenate([x.astype(f32), jnp.zeros((_NP - _N, _D), f32)])
    seg3 = jnp.concatenate([batch, jnp.zeros((_NP - _N,), jnp.int32)])
    seg3 = seg3.reshape(_NP // _BN, 1, _BN)
    zr = jnp.zeros((_CH, _D), f32)

    def f16i(t):
        t = jnp.where(jnp.abs(t) < 6.104e-5, 0.0, t)
        b = t.astype(jnp.float16).reshape(_NP, _D // 32, 2, 16)
        b = b.transpose(0, 1, 3, 2).reshape(_NP, _D // 2, 2)
        return lax.bitcast_convert_type(b, jnp.int32)

    agg1 = _sc_segment_sum(xp, sd3, zr)
    h1, st1 = _mlp(xp, agg1, W1_1, b1_1, W1_2, b1_2)
    h1n = _norm(h1, st1, gamma1, beta1)
    agg2 = _sc_segment_sum(h1n, sd3, zr)
    h2, st2 = _mlp(h1n, agg2, W2_1, b2_1, W2_2, b2_2)
    h2n = _norm(h2, st2, gamma2, beta2)
    agg3 = _sc_segment_sum(h2n, sd3, zr)
    h3, st3 = _mlp(h2n, agg3, W3_1, b3_1, W3_2, b3_2)
    return _pool(h3, st3, gamma3, beta3, seg3)
